# baseline (speedup 1.0000x reference)
import jax
import jax.numpy as jnp
from jax import lax
from jax.experimental import pallas as pl
from jax.experimental.pallas import tpu as pltpu

VOCAB = 1000
D_POS = 128
D_OUT = VOCAB + D_POS
BATCH_TILE = 16
NBUF = 4


def _body(x_ref, pos_ref, out_hbm, scratch, sems):
    i = pl.program_id(0)
    n = pl.num_programs(0)
    bt = BATCH_TILE
    l = pos_ref.shape[0]

    for s in range(NBUF):

        @pl.when(lax.rem(i, NBUF) == s)
        def _():
            @pl.when(i >= NBUF)
            def _():
                pltpu.make_async_copy(
                    scratch.at[s], out_hbm.at[pl.ds(0, bt)], sems.at[s]
                ).wait()

            x = x_ref[...]
            lanes = lax.broadcasted_iota(jnp.int32, (bt, l, D_OUT), 2)
            pos_b = jnp.broadcast_to(pos_ref[...][None, :, :], (bt, l, D_OUT))
            scratch[s] = jnp.where(lanes == x[:, :, None], jnp.float32(1.0), pos_b)
            pltpu.make_async_copy(
                scratch.at[s], out_hbm.at[pl.ds(i * bt, bt)], sems.at[s]
            ).start()

    @pl.when(i == n - 1)
    def _():
        for s in range(NBUF):
            pltpu.make_async_copy(
                scratch.at[s], out_hbm.at[pl.ds(0, bt)], sems.at[s]
            ).wait()


def kernel(X, position_embeddings):
    batch, length = X.shape
    pos_pad = jnp.pad(position_embeddings, ((0, 0), (VOCAB, 0)))
    grid = (batch // BATCH_TILE,)
    return pl.pallas_call(
        _body,
        grid=grid,
        in_specs=[
            pl.BlockSpec((BATCH_TILE, length), lambda i: (i, 0)),
            pl.BlockSpec((length, D_OUT), lambda i: (0, 0)),
        ],
        out_specs=pl.BlockSpec(memory_space=pl.ANY),
        out_shape=jax.ShapeDtypeStruct((batch, length, D_OUT), jnp.float32),
        scratch_shapes=[
            pltpu.VMEM((NBUF, BATCH_TILE, length, D_OUT), jnp.float32),
            pltpu.SemaphoreType.DMA((NBUF,)),
        ],
    )(X, pos_pad)
